# Initial kernel scaffold; baseline (speedup 1.0000x reference)
import jax
import jax.numpy as jnp
from jax.experimental import pallas as pl


def kernel(sp_fea, sp_xyz, o_p_fea, p_xyz, c2p_idx_abs, c2p_idx, cluster_idx, mlp_W1, mlp_g1, mlp_b1, mlp_W2, mlp_g2, mlp_b2, wfea_W1, wfea_g1, wfea_b1, wfea_W2, wfea_g2, wfea_b2, wxyz_W1, wxyz_g1, wxyz_b1, wxyz_W2, wxyz_g2, wxyz_b2):
    raise NotImplementedError("write your pallas kernel here")


# speedup vs baseline: 7.7657x; 7.7657x over previous
import functools

import jax
import jax.numpy as jnp
from jax import lax
from jax.experimental import pallas as pl
from jax.experimental.pallas import tpu as pltpu
from jax.experimental.pallas import tpu_sc as plsc

N = 100000
M = 1000
K = 6
C = 32
H = 16
INV = (1.0 + 1e-5) ** -0.5

NW = 32
NP = 102400
NPK = NP * K
TN = 512

GB_STREAM = 128
GB_NSTREAM = 10
GB_CHUNK = GB_STREAM * GB_NSTREAM
GB_NCHUNK = NPK // (NW * GB_CHUNK)

SD_PTS = 128
SD_PAIRS = SD_PTS * K
SD_NCHUNK = NP // (NW * SD_PTS)
MA = 1024


def _tables_body(spf_ref, spx_ref, wf_ref, wx_ref, out_ref):
    out_ref[:, 0:16] = jnp.dot(spf_ref[...], wf_ref[...],
                               preferred_element_type=jnp.float32)
    out_ref[:, 16:32] = jnp.dot(spx_ref[...], wx_ref[...],
                                preferred_element_type=jnp.float32)


def _project_tables(spf, spx, wf1t, wx1t):
    return pl.pallas_call(
        _tables_body,
        out_shape=jax.ShapeDtypeStruct((M, 2 * H), jnp.float32),
    )(spf, spx, wf1t, wx1t)


def _gather_body(table_ref, idx_ref, out_ref, idx_v, rows_v, sem):
    wid = lax.axis_index("s") * 2 + lax.axis_index("c")

    def chunk(ci, carry):
        base = wid * (GB_NCHUNK * GB_CHUNK) + ci * GB_CHUNK
        for si in range(GB_NSTREAM):
            pltpu.sync_copy(idx_ref.at[pl.ds(base + si * GB_STREAM, GB_STREAM)],
                            idx_v.at[si])
        cps = [
            pltpu.async_copy(table_ref.at[idx_v.at[si]],
                             rows_v.at[pl.ds(si * GB_STREAM, GB_STREAM)], sem)
            for si in range(GB_NSTREAM)
        ]
        for cp in cps:
            cp.wait()
        pltpu.sync_copy(rows_v, out_ref.at[pl.ds(base, GB_CHUNK)])
        return carry

    lax.fori_loop(0, GB_NCHUNK, chunk, 0)


def _sc_gather(table, idx2d):
    mesh = plsc.VectorSubcoreMesh(core_axis_name="c", subcore_axis_name="s")
    f = functools.partial(
        pl.kernel,
        mesh=mesh,
        out_type=jax.ShapeDtypeStruct((NPK, 2 * H), jnp.float32),
        scratch_types=[
            pltpu.VMEM((GB_NSTREAM, GB_STREAM), jnp.int32),
            pltpu.VMEM((GB_CHUNK, 2 * H), jnp.float32),
            pltpu.SemaphoreType.DMA,
        ],
        compiler_params=pltpu.CompilerParams(use_tc_tiling_on_sc=False),
    )(_gather_body)
    return f(table, idx2d)


def _dense_body(g_ref, pv_ref, wm_ref, mg1, mb1, mw2t, mg2, mb2,
                wfp, fg1, fb1, fw2t, fg2, fb2,
                wxp, xg1, xb1, xw2t, xg2, xb2, out_ref):
    pv = pv_ref[...]
    h = jnp.maximum(jnp.dot(pv, wm_ref[...], preferred_element_type=jnp.float32)
                    * INV * mg1[...] + mb1[...], 0.0)
    h = jnp.maximum(jnp.dot(h, mw2t[...], preferred_element_type=jnp.float32)
                    * INV * mg2[...] + mb2[...], 0.0)
    pn = h / jnp.maximum(jnp.sqrt(jnp.sum(h * h, axis=-1, keepdims=True)), 1e-12)

    ofp = jnp.dot(pv, wfp[...], preferred_element_type=jnp.float32)
    pxp = jnp.dot(pv, wxp[...], preferred_element_type=jnp.float32)

    g = g_ref[...]
    logits = []
    for k in range(K):
        gf = g[:, k * 32:k * 32 + 16]
        gx = g[:, k * 32 + 16:k * 32 + 32]
        hf = jnp.maximum((gf - ofp) * INV * fg1[...] + fb1[...], 0.0)
        hf = jnp.maximum(jnp.dot(hf, fw2t[...], preferred_element_type=jnp.float32)
                         * INV * fg2[...] + fb2[...], 0.0)
        nf = jnp.maximum(jnp.sqrt(jnp.sum(hf * hf, axis=-1, keepdims=True)), 1e-12)
        af = jnp.sum(pn * hf, axis=-1, keepdims=True) / nf

        hx = jnp.maximum((gx - pxp) * INV * xg1[...] + xb1[...], 0.0)
        hx = jnp.maximum(jnp.dot(hx, xw2t[...], preferred_element_type=jnp.float32)
                         * INV * xg2[...] + xb2[...], 0.0)
        nx = jnp.maximum(jnp.sqrt(jnp.sum(hx * hx, axis=-1, keepdims=True)), 1e-12)
        ax = jnp.sum(pn * hx, axis=-1, keepdims=True) / nx
        logits.append(af * ax)

    lg = jnp.concatenate(logits, axis=-1)
    mx = jnp.max(lg, axis=-1, keepdims=True)
    e = jnp.exp(lg - mx)
    w = e / jnp.sum(e, axis=-1, keepdims=True)
    row = pl.program_id(0) * TN + lax.broadcasted_iota(jnp.int32, (TN, 1), 0)
    out_ref[...] = w * (row < N).astype(jnp.float32)


def _dense_stage(g, pv, wm, mg1, mb1, mw2t, mg2, mb2,
                 wfp, fg1, fb1, fw2t, fg2, fb2,
                 wxp, xg1, xb1, xw2t, xg2, xb2):
    full = lambda shape: pl.BlockSpec(shape, lambda i: (0, 0))
    return pl.pallas_call(
        _dense_body,
        grid=(NP // TN,),
        in_specs=[
            pl.BlockSpec((TN, K * C), lambda i: (i, 0)),
            pl.BlockSpec((TN, 48), lambda i: (i, 0)),
            full((48, H)), full((1, H)), full((1, H)),
            full((H, H)), full((1, H)), full((1, H)),
            full((48, H)), full((1, H)), full((1, H)),
            full((H, H)), full((1, H)), full((1, H)),
            full((48, H)), full((1, H)), full((1, H)),
            full((H, H)), full((1, H)), full((1, H)),
        ],
        out_specs=pl.BlockSpec((TN, K), lambda i: (i, 0)),
        out_shape=jax.ShapeDtypeStruct((NP, K), jnp.float32),
    )(g, pv, wm, mg1, mb1, mw2t, mg2, mb2,
      wfp, fg1, fb1, fw2t, fg2, fb2,
      wxp, xg1, xb1, xw2t, xg2, xb2)


def _scatter_body(w_ref, idx_ref, pv_ref, out_ref,
                  pvb, wb, idxb, payl, zb, acc):
    c = lax.axis_index("c")
    s = lax.axis_index("s")
    wid = s * 2 + c

    def zrow(j, carry):
        for p in range(3):
            zb[j, pl.ds(p * 16, 16)] = jnp.zeros((16,), jnp.float32)
        return carry

    lax.fori_loop(0, MA // 16, zrow, 0)
    pltpu.sync_copy(zb, acc.at[pl.ds(s * (MA // 16), MA // 16)])
    plsc.subcore_barrier()

    def chunk(ci, carry):
        pt0 = wid * (SD_NCHUNK * SD_PTS) + ci * SD_PTS
        pltpu.sync_copy(pv_ref.at[pl.ds(pt0, SD_PTS)], pvb)
        pltpu.sync_copy(w_ref.at[pl.ds(pt0 * K, SD_PAIRS)], wb)
        for si in range(K):
            pltpu.sync_copy(idx_ref.at[pl.ds(pt0 * K + si * 128, 128)],
                            idxb.at[si])

        def point(j, carry2):
            pv0 = pvb[j, pl.ds(0, 16)]
            pv1 = pvb[j, pl.ds(16, 16)]
            pv2 = pvb[j, pl.ds(32, 16)]
            for k in range(K):
                wv = plsc.load_gather(
                    wb, [jnp.full((16,), j * K + k, jnp.int32)])
                payl[j * K + k, pl.ds(0, 16)] = wv * pv0
                payl[j * K + k, pl.ds(16, 16)] = wv * pv1
                payl[j * K + k, pl.ds(32, 16)] = wv * pv2
            return carry2

        lax.fori_loop(0, SD_PTS, point, 0)
        for si in range(K):
            pltpu.sync_copy(payl.at[pl.ds(si * 128, 128)],
                            acc.at[idxb.at[si]], add=True)
        return carry

    lax.fori_loop(0, SD_NCHUNK, chunk, 0)
    plsc.subcore_barrier()

    @pl.when(s == 0)
    def _():
        pltpu.sync_copy(acc, out_ref.at[c])


def _sc_scatter(w_flat, idx2d, pv):
    mesh = plsc.VectorSubcoreMesh(core_axis_name="c", subcore_axis_name="s")
    f = functools.partial(
        pl.kernel,
        mesh=mesh,
        out_type=jax.ShapeDtypeStruct((2, MA, 48), jnp.float32),
        scratch_types=[
            pltpu.VMEM((SD_PTS, 48), jnp.float32),
            pltpu.VMEM((SD_PAIRS,), jnp.float32),
            pltpu.VMEM((K, 128), jnp.int32),
            pltpu.VMEM((SD_PAIRS, 48), jnp.float32),
            pltpu.VMEM((MA // 16, 48), jnp.float32),
            pltpu.VMEM_SHARED((MA, 48), jnp.float32),
        ],
        compiler_params=pltpu.CompilerParams(use_tc_tiling_on_sc=False,
                                             needs_layout_passes=False),
    )(_scatter_body)
    return f(w_flat, idx2d, pv)


def kernel(sp_fea, sp_xyz, o_p_fea, p_xyz, c2p_idx_abs, c2p_idx, cluster_idx,
           mlp_W1, mlp_g1, mlp_b1, mlp_W2, mlp_g2, mlp_b2,
           wfea_W1, wfea_g1, wfea_b1, wfea_W2, wfea_g2, wfea_b2,
           wxyz_W1, wxyz_g1, wxyz_b1, wxyz_W2, wxyz_g2, wxyz_b2):
    f32 = jnp.float32
    spf, spx = sp_fea[0], sp_xyz[0]
    opf, pxy = o_p_fea[0], p_xyz[0]

    pad_pairs = NPK - N * K
    idxg = jnp.concatenate(
        [c2p_idx_abs[0].reshape(-1).astype(jnp.int32),
         jnp.zeros((pad_pairs,), jnp.int32)])
    idxs = jnp.concatenate(
        [c2p_idx[0].reshape(-1).astype(jnp.int32),
         jnp.zeros((pad_pairs,), jnp.int32)])

    pv = jnp.concatenate(
        [opf, pxy, jnp.ones((N, 1), f32), jnp.zeros((N, 12), f32)], axis=1)
    pv = jnp.concatenate([pv, jnp.zeros((NP - N, 48), f32)], axis=0)

    wm = jnp.zeros((48, H), f32).at[0:C].set(mlp_W1.T)
    wfp = jnp.zeros((48, H), f32).at[0:C].set(wfea_W1.T)
    wxp = jnp.zeros((48, H), f32).at[32:35].set(wxyz_W1.T)
    r = lambda v: v.reshape(1, H)

    tables = _project_tables(spf, spx, wfea_W1.T, wxyz_W1.T)
    g = _sc_gather(tables, idxg)
    w = _dense_stage(
        g.reshape(NP, K * C), pv,
        wm, r(mlp_g1), r(mlp_b1), mlp_W2.T, r(mlp_g2), r(mlp_b2),
        wfp, r(wfea_g1), r(wfea_b1), wfea_W2.T, r(wfea_g2), r(wfea_b2),
        wxp, r(wxyz_g1), r(wxyz_b1), wxyz_W2.T, r(wxyz_g2), r(wxyz_b2))
    part = _sc_scatter(w.reshape(-1), idxs, pv)

    acc = part[0, :M] + part[1, :M]
    denom = acc[:, 35:36] + 1e-8
    return (acc[:, 0:C] / denom)[None], (acc[:, 32:35] / denom)[None]


# speedup vs baseline: 15.2144x; 1.9592x over previous
import functools

import jax
import jax.numpy as jnp
from jax import lax
from jax.experimental import pallas as pl
from jax.experimental.pallas import tpu as pltpu
from jax.experimental.pallas import tpu_sc as plsc

N = 100000
M = 1000
K = 6
C = 32
H = 16
INV = (1.0 + 1e-5) ** -0.5

NW = 32
NP = 102400
NPK = NP * K
TN = 512

GB_STREAM = 128
GB_NSTREAM = 10
GB_CHUNK = GB_STREAM * GB_NSTREAM
GB_NCHUNK = NPK // (NW * GB_CHUNK)

SD_PTS = 128
SD_PAIRS = SD_PTS * K
SD_NCHUNK = NP // (NW * SD_PTS)
MA = 1024


def _tables_body(spf_ref, spx_ref, wf_ref, wx_ref, out_ref):
    out_ref[:, 0:16] = jnp.dot(spf_ref[...], wf_ref[...],
                               preferred_element_type=jnp.float32)
    out_ref[:, 16:32] = jnp.dot(spx_ref[...], wx_ref[...],
                                preferred_element_type=jnp.float32)


def _project_tables(spf, spx, wf1t, wx1t):
    return pl.pallas_call(
        _tables_body,
        out_shape=jax.ShapeDtypeStruct((M, 2 * H), jnp.float32),
    )(spf, spx, wf1t, wx1t)


def _gather_body(table_ref, idx_ref, out_ref, idx_all, rows_a, rows_b,
                 gsem, wsem_a, wsem_b):
    wid = lax.axis_index("s") * 2 + lax.axis_index("c")
    base = wid * (GB_NCHUNK * GB_CHUNK)
    pltpu.sync_copy(idx_ref.at[pl.ds(base, GB_NCHUNK * GB_CHUNK)], idx_all)

    bufs = [rows_a, rows_b]
    wsems = [wsem_a, wsem_b]
    pending = [None, None]
    for ci in range(GB_NCHUNK):
        b = ci % 2
        if pending[b] is not None:
            pending[b].wait()
        cps = [
            pltpu.async_copy(
                table_ref.at[idx_all.at[pl.ds((ci * GB_NSTREAM + si) * GB_STREAM,
                                              GB_STREAM)]],
                bufs[b].at[pl.ds(si * GB_STREAM, GB_STREAM)], gsem)
            for si in range(GB_NSTREAM)
        ]
        for cp in cps:
            cp.wait()
        pending[b] = pltpu.async_copy(
            bufs[b], out_ref.at[pl.ds(base + ci * GB_CHUNK, GB_CHUNK)], wsems[b])
    for p in pending:
        if p is not None:
            p.wait()


def _sc_gather(table, idx2d):
    mesh = plsc.VectorSubcoreMesh(core_axis_name="c", subcore_axis_name="s")
    f = functools.partial(
        pl.kernel,
        mesh=mesh,
        out_type=jax.ShapeDtypeStruct((NPK, 2 * H), jnp.float32),
        scratch_types=[
            pltpu.VMEM((GB_NCHUNK * GB_CHUNK,), jnp.int32),
            pltpu.VMEM((GB_CHUNK, 2 * H), jnp.float32),
            pltpu.VMEM((GB_CHUNK, 2 * H), jnp.float32),
            pltpu.SemaphoreType.DMA,
            pltpu.SemaphoreType.DMA,
            pltpu.SemaphoreType.DMA,
        ],
        compiler_params=pltpu.CompilerParams(use_tc_tiling_on_sc=False),
    )(_gather_body)
    return f(table, idx2d)


def _dense_body(g_ref, pv_ref, wm_ref, mg1, mb1, mw2t, mg2, mb2,
                w1big, g1big, b1big, w2big, g2big, b2big, sel, out_ref):
    pv = pv_ref[...]
    h = jnp.maximum(jnp.dot(pv, wm_ref[...], preferred_element_type=jnp.float32)
                    * INV * mg1[...] + mb1[...], 0.0)
    h = jnp.maximum(jnp.dot(h, mw2t[...], preferred_element_type=jnp.float32)
                    * INV * mg2[...] + mb2[...], 0.0)
    pn = h / jnp.maximum(jnp.sqrt(jnp.sum(h * h, axis=-1, keepdims=True)), 1e-12)

    opx6 = jnp.dot(pv, w1big[...], preferred_element_type=jnp.float32)
    h1 = jnp.maximum((g_ref[...] - opx6) * INV * g1big[...] + b1big[...], 0.0)
    h2 = jnp.dot(h1, w2big[...], preferred_element_type=jnp.float32)
    h2 = jnp.maximum(h2 * INV * g2big[...] + b2big[...], 0.0)

    pn12 = jnp.concatenate([pn] * (2 * K), axis=1)
    sq = jnp.dot(h2 * h2, sel[...], preferred_element_type=jnp.float32)
    dt = jnp.dot(h2 * pn12, sel[...], preferred_element_type=jnp.float32)
    cos = dt / jnp.maximum(jnp.sqrt(sq), 1e-12)
    lg = cos[:, 0:K] * cos[:, K:2 * K]

    mx = jnp.max(lg, axis=-1, keepdims=True)
    e = jnp.exp(lg - mx)
    w = e / jnp.sum(e, axis=-1, keepdims=True)
    row = pl.program_id(0) * TN + lax.broadcasted_iota(jnp.int32, (TN, 1), 0)
    out_ref[...] = w * (row < N).astype(jnp.float32)


def _dense_stage(g, pv, wm, mg1, mb1, mw2t, mg2, mb2,
                 w1big, g1big, b1big, w2big, g2big, b2big, sel):
    full = lambda shape: pl.BlockSpec(shape, lambda i: (0, 0))
    return pl.pallas_call(
        _dense_body,
        grid=(NP // TN,),
        in_specs=[
            pl.BlockSpec((TN, K * C), lambda i: (i, 0)),
            pl.BlockSpec((TN, 48), lambda i: (i, 0)),
            full((48, H)), full((1, H)), full((1, H)),
            full((H, H)), full((1, H)), full((1, H)),
            full((48, K * C)), full((1, K * C)), full((1, K * C)),
            full((K * C, K * C)), full((1, K * C)), full((1, K * C)),
            full((K * C, 2 * K)),
        ],
        out_specs=pl.BlockSpec((TN, K), lambda i: (i, 0)),
        out_shape=jax.ShapeDtypeStruct((NP, K), jnp.float32),
    )(g, pv, wm, mg1, mb1, mw2t, mg2, mb2,
      w1big, g1big, b1big, w2big, g2big, b2big, sel)


def _scatter_body(w_ref, idx_ref, pv_ref, out_ref,
                  pvb_a, pvb_b, wb_a, wb_b, idxb_a, idxb_b, idxb_c,
                  payl_a, payl_b, zb, acc,
                  lsem_a, lsem_b, ssem_a, ssem_b):
    c = lax.axis_index("c")
    s = lax.axis_index("s")
    wid = s * 2 + c

    def zrow(j, carry):
        for p in range(3):
            zb[j, pl.ds(p * 16, 16)] = jnp.zeros((16,), jnp.float32)
        return carry

    lax.fori_loop(0, MA // 16, zrow, 0)
    pltpu.sync_copy(zb, acc.at[pl.ds(s * (MA // 16), MA // 16)])
    plsc.subcore_barrier()

    pvb = [pvb_a, pvb_b]
    wb = [wb_a, wb_b]
    idxb = [idxb_a, idxb_b, idxb_c]
    payl = [payl_a, payl_b]
    lsem = [lsem_a, lsem_b]
    ssem = [ssem_a, ssem_b]

    def load(ci):
        b = ci % 2
        pt0 = wid * (SD_NCHUNK * SD_PTS) + ci * SD_PTS
        cps = [pltpu.async_copy(pv_ref.at[pl.ds(pt0, SD_PTS)], pvb[b], lsem[b]),
               pltpu.async_copy(w_ref.at[pl.ds(pt0 * K, SD_PAIRS)], wb[b],
                                lsem[b])]
        cps += [
            pltpu.async_copy(idx_ref.at[pl.ds(pt0 * K + si * 128, 128)],
                             idxb[ci % 3].at[si], lsem[b])
            for si in range(K)
        ]
        return cps

    loads = [None, None]
    scats = [None, None]
    loads[0] = load(0)
    for ci in range(SD_NCHUNK):
        b = ci % 2
        if ci + 1 < SD_NCHUNK:
            loads[1 - b] = load(ci + 1)
        for cp in loads[b]:
            cp.wait()
        if scats[b] is not None:
            for cp in scats[b]:
                cp.wait()

        def point(j, carry2, _pvb=pvb[b], _wb=wb[b], _payl=payl[b]):
            pv0 = _pvb[j, pl.ds(0, 16)]
            pv1 = _pvb[j, pl.ds(16, 16)]
            pv2 = _pvb[j, pl.ds(32, 16)]
            for k in range(K):
                wv = plsc.load_gather(
                    _wb, [jnp.full((16,), j * K + k, jnp.int32)])
                _payl[j * K + k, pl.ds(0, 16)] = wv * pv0
                _payl[j * K + k, pl.ds(16, 16)] = wv * pv1
                _payl[j * K + k, pl.ds(32, 16)] = wv * pv2
            return carry2

        lax.fori_loop(0, SD_PTS, point, 0)
        scats[b] = [
            pltpu.async_copy(payl[b].at[pl.ds(si * 128, 128)],
                             acc.at[idxb[ci % 3].at[si]], ssem[b], add=True)
            for si in range(K)
        ]
    for pend in scats:
        if pend is not None:
            for cp in pend:
                cp.wait()
    plsc.subcore_barrier()

    @pl.when(s == 0)
    def _():
        pltpu.sync_copy(acc, out_ref.at[c])


def _sc_scatter(w_flat, idx2d, pv):
    mesh = plsc.VectorSubcoreMesh(core_axis_name="c", subcore_axis_name="s")
    f = functools.partial(
        pl.kernel,
        mesh=mesh,
        out_type=jax.ShapeDtypeStruct((2, MA, 48), jnp.float32),
        scratch_types=[
            pltpu.VMEM((SD_PTS, 48), jnp.float32),
            pltpu.VMEM((SD_PTS, 48), jnp.float32),
            pltpu.VMEM((SD_PAIRS,), jnp.float32),
            pltpu.VMEM((SD_PAIRS,), jnp.float32),
            pltpu.VMEM((K, 128), jnp.int32),
            pltpu.VMEM((K, 128), jnp.int32),
            pltpu.VMEM((K, 128), jnp.int32),
            pltpu.VMEM((SD_PAIRS, 48), jnp.float32),
            pltpu.VMEM((SD_PAIRS, 48), jnp.float32),
            pltpu.VMEM((MA // 16, 48), jnp.float32),
            pltpu.VMEM_SHARED((MA, 48), jnp.float32),
            pltpu.SemaphoreType.DMA,
            pltpu.SemaphoreType.DMA,
            pltpu.SemaphoreType.DMA,
            pltpu.SemaphoreType.DMA,
        ],
        compiler_params=pltpu.CompilerParams(use_tc_tiling_on_sc=False,
                                             needs_layout_passes=False),
    )(_scatter_body)
    return f(w_flat, idx2d, pv)


def kernel(sp_fea, sp_xyz, o_p_fea, p_xyz, c2p_idx_abs, c2p_idx, cluster_idx,
           mlp_W1, mlp_g1, mlp_b1, mlp_W2, mlp_g2, mlp_b2,
           wfea_W1, wfea_g1, wfea_b1, wfea_W2, wfea_g2, wfea_b2,
           wxyz_W1, wxyz_g1, wxyz_b1, wxyz_W2, wxyz_g2, wxyz_b2):
    f32 = jnp.float32
    spf, spx = sp_fea[0], sp_xyz[0]
    opf, pxy = o_p_fea[0], p_xyz[0]

    pad_pairs = NPK - N * K
    idxg = jnp.concatenate(
        [c2p_idx_abs[0].reshape(-1).astype(jnp.int32),
         jnp.zeros((pad_pairs,), jnp.int32)])
    idxs = jnp.concatenate(
        [c2p_idx[0].reshape(-1).astype(jnp.int32),
         jnp.zeros((pad_pairs,), jnp.int32)])

    pv = jnp.concatenate(
        [opf, pxy, jnp.ones((N, 1), f32), jnp.zeros((N, 12), f32)], axis=1)
    pv = jnp.concatenate([pv, jnp.zeros((NP - N, 48), f32)], axis=0)

    wm = jnp.zeros((48, H), f32).at[0:C].set(mlp_W1.T)
    wfp = jnp.zeros((48, H), f32).at[0:C].set(wfea_W1.T)
    wxp = jnp.zeros((48, H), f32).at[32:35].set(wxyz_W1.T)
    r = lambda v: v.reshape(1, H)

    KC = K * C
    w1big = jnp.zeros((48, KC), f32)
    w2big = jnp.zeros((KC, KC), f32)
    sel = jnp.zeros((KC, 2 * K), f32)
    for k in range(K):
        w1big = w1big.at[:, k * 32:k * 32 + 16].set(wfp)
        w1big = w1big.at[:, k * 32 + 16:k * 32 + 32].set(wxp)
        w2big = w2big.at[k * 32:k * 32 + 16, k * 32:k * 32 + 16].set(wfea_W2.T)
        w2big = w2big.at[k * 32 + 16:k * 32 + 32,
                         k * 32 + 16:k * 32 + 32].set(wxyz_W2.T)
        sel = sel.at[k * 32:k * 32 + 16, k].set(1.0)
        sel = sel.at[k * 32 + 16:k * 32 + 32, K + k].set(1.0)
    gx1 = jnp.concatenate([wfea_g1, wxyz_g1])
    bx1 = jnp.concatenate([wfea_b1, wxyz_b1])
    gx2 = jnp.concatenate([wfea_g2, wxyz_g2])
    bx2 = jnp.concatenate([wfea_b2, wxyz_b2])
    tile6 = lambda v: jnp.tile(v, (K,)).reshape(1, KC)

    tables = _project_tables(spf, spx, wfea_W1.T, wxyz_W1.T)
    g = _sc_gather(tables, idxg)
    w = _dense_stage(
        g.reshape(NP, K * C), pv,
        wm, r(mlp_g1), r(mlp_b1), mlp_W2.T, r(mlp_g2), r(mlp_b2),
        w1big, tile6(gx1), tile6(bx1), w2big, tile6(gx2), tile6(bx2), sel)
    part = _sc_scatter(w.reshape(-1), idxs, pv)

    acc = part[0, :M] + part[1, :M]
    denom = acc[:, 35:36] + 1e-8
    return (acc[:, 0:C] / denom)[None], (acc[:, 32:35] / denom)[None]


# speedup vs baseline: 17.9666x; 1.1809x over previous
import functools

import jax
import jax.numpy as jnp
from jax import lax
from jax.experimental import pallas as pl
from jax.experimental.pallas import tpu as pltpu
from jax.experimental.pallas import tpu_sc as plsc

N = 100000
M = 1000
K = 6
C = 32
H = 16
INV = (1.0 + 1e-5) ** -0.5

NW = 32
NP = 102400
NPK = NP * K
TN = 512

GB_CHUNK = 640
GB_NCHUNK = NPK // (NW * GB_CHUNK)

SD_PTS = 128
SD_PAIRS = SD_PTS * K
SD_NCHUNK = NP // (NW * SD_PTS)
MA = 1024


def _tables_body(spf_ref, spx_ref, wf_ref, wx_ref, out_ref):
    out_ref[:, 0:16] = jnp.dot(spf_ref[...], wf_ref[...],
                               preferred_element_type=jnp.float32)
    out_ref[:, 16:32] = jnp.dot(spx_ref[...], wx_ref[...],
                                preferred_element_type=jnp.float32)


def _project_tables(spf, spx, wf1t, wx1t):
    return pl.pallas_call(
        _tables_body,
        out_shape=jax.ShapeDtypeStruct((M, 2 * H), jnp.float32),
    )(spf, spx, wf1t, wx1t)


def _gather_body(table_ref, idx_ref, out_ref, tb, idx_a, idx_b,
                 rows_a, rows_b, lsem_a, lsem_b, wsem_a, wsem_b):
    wid = lax.axis_index("s") * 2 + lax.axis_index("c")
    base = wid * (GB_NCHUNK * GB_CHUNK)
    pltpu.sync_copy(table_ref, tb)

    idxb = [idx_a, idx_b]
    rows = [rows_a, rows_b]
    lsem = [lsem_a, lsem_b]
    wsem = [wsem_a, wsem_b]
    iota = lax.broadcasted_iota(jnp.int32, (16,), 0)

    def load(ci):
        return pltpu.async_copy(
            idx_ref.at[pl.ds(base + ci * GB_CHUNK, GB_CHUNK)],
            idxb[ci % 2], lsem[ci % 2])

    loads = [load(0), load(1)]
    wpend = [None, None]
    for ci in range(GB_NCHUNK):
        b = ci % 2
        loads[b].wait()
        if wpend[b] is not None:
            wpend[b].wait()

        def group(g, carry, _idx=idxb[b], _rows=rows[b]):
            m_vec = _idx[pl.ds(g * 16, 16)]
            for jj in range(16):
                sel16 = jnp.full((16,), jj, jnp.int32)
                m_b = m_vec.at[sel16].get(mode="promise_in_bounds")
                a0 = m_b * 32 + iota
                r0 = plsc.load_gather(tb, [a0])
                r1 = plsc.load_gather(tb, [a0 + 16])
                _rows[pl.ds(g * 512 + jj * 32, 16)] = r0
                _rows[pl.ds(g * 512 + jj * 32 + 16, 16)] = r1
            return carry

        lax.fori_loop(0, GB_CHUNK // 16, group, 0)
        if ci + 2 < GB_NCHUNK:
            loads[b] = load(ci + 2)
        wpend[b] = pltpu.async_copy(
            rows[b], out_ref.at[pl.ds((base + ci * GB_CHUNK) * 32,
                                      GB_CHUNK * 32)], wsem[b])
    for p in wpend:
        if p is not None:
            p.wait()


def _sc_gather(table, idx1d):
    mesh = plsc.VectorSubcoreMesh(core_axis_name="c", subcore_axis_name="s")
    f = functools.partial(
        pl.kernel,
        mesh=mesh,
        out_type=jax.ShapeDtypeStruct((NPK * 2 * H,), jnp.float32),
        scratch_types=[
            pltpu.VMEM((M * 2 * H,), jnp.float32),
            pltpu.VMEM((GB_CHUNK,), jnp.int32),
            pltpu.VMEM((GB_CHUNK,), jnp.int32),
            pltpu.VMEM((GB_CHUNK * 2 * H,), jnp.float32),
            pltpu.VMEM((GB_CHUNK * 2 * H,), jnp.float32),
            pltpu.SemaphoreType.DMA,
            pltpu.SemaphoreType.DMA,
            pltpu.SemaphoreType.DMA,
            pltpu.SemaphoreType.DMA,
        ],
        compiler_params=pltpu.CompilerParams(use_tc_tiling_on_sc=False,
                                             needs_layout_passes=False),
    )(_gather_body)
    return f(table, idx1d)


def _dense_body(g_ref, pv_ref, wm_ref, mg1, mb1, mw2t, mg2, mb2,
                w1big, g1big, b1big, w2big, g2big, b2big, sel, out_ref):
    pv = pv_ref[...]
    h = jnp.maximum(jnp.dot(pv, wm_ref[...], preferred_element_type=jnp.float32)
                    * INV * mg1[...] + mb1[...], 0.0)
    h = jnp.maximum(jnp.dot(h, mw2t[...], preferred_element_type=jnp.float32)
                    * INV * mg2[...] + mb2[...], 0.0)
    pn = h / jnp.maximum(jnp.sqrt(jnp.sum(h * h, axis=-1, keepdims=True)), 1e-12)

    opx6 = jnp.dot(pv, w1big[...], preferred_element_type=jnp.float32)
    h1 = jnp.maximum((g_ref[...] - opx6) * INV * g1big[...] + b1big[...], 0.0)
    h2 = jnp.dot(h1, w2big[...], preferred_element_type=jnp.float32)
    h2 = jnp.maximum(h2 * INV * g2big[...] + b2big[...], 0.0)

    pn12 = jnp.concatenate([pn] * (2 * K), axis=1)
    sq = jnp.dot(h2 * h2, sel[...], preferred_element_type=jnp.float32)
    dt = jnp.dot(h2 * pn12, sel[...], preferred_element_type=jnp.float32)
    cos = dt / jnp.maximum(jnp.sqrt(sq), 1e-12)
    lg = cos[:, 0:K] * cos[:, K:2 * K]

    mx = jnp.max(lg, axis=-1, keepdims=True)
    e = jnp.exp(lg - mx)
    w = e / jnp.sum(e, axis=-1, keepdims=True)
    row = pl.program_id(0) * TN + lax.broadcasted_iota(jnp.int32, (TN, 1), 0)
    out_ref[...] = w * (row < N).astype(jnp.float32)


def _dense_stage(g, pv, wm, mg1, mb1, mw2t, mg2, mb2,
                 w1big, g1big, b1big, w2big, g2big, b2big, sel):
    full = lambda shape: pl.BlockSpec(shape, lambda i: (0, 0))
    return pl.pallas_call(
        _dense_body,
        grid=(NP // TN,),
        in_specs=[
            pl.BlockSpec((TN, K * C), lambda i: (i, 0)),
            pl.BlockSpec((TN, 48), lambda i: (i, 0)),
            full((48, H)), full((1, H)), full((1, H)),
            full((H, H)), full((1, H)), full((1, H)),
            full((48, K * C)), full((1, K * C)), full((1, K * C)),
            full((K * C, K * C)), full((1, K * C)), full((1, K * C)),
            full((K * C, 2 * K)),
        ],
        out_specs=pl.BlockSpec((TN, K), lambda i: (i, 0)),
        out_shape=jax.ShapeDtypeStruct((NP, K), jnp.float32),
    )(g, pv, wm, mg1, mb1, mw2t, mg2, mb2,
      w1big, g1big, b1big, w2big, g2big, b2big, sel)


def _scatter_body(w_ref, idx_ref, pv_ref, out_ref,
                  pvb_a, pvb_b, wb_a, wb_b, idxb_a, idxb_b, idxb_c,
                  payl_a, payl_b, zb, acc,
                  lsem_a, lsem_b, ssem_a, ssem_b):
    c = lax.axis_index("c")
    s = lax.axis_index("s")
    wid = s * 2 + c

    def zrow(j, carry):
        for p in range(3):
            zb[j, pl.ds(p * 16, 16)] = jnp.zeros((16,), jnp.float32)
        return carry

    lax.fori_loop(0, MA // 16, zrow, 0)
    pltpu.sync_copy(zb, acc.at[pl.ds(s * (MA // 16), MA // 16)])
    plsc.subcore_barrier()

    pvb = [pvb_a, pvb_b]
    wb = [wb_a, wb_b]
    idxb = [idxb_a, idxb_b, idxb_c]
    payl = [payl_a, payl_b]
    lsem = [lsem_a, lsem_b]
    ssem = [ssem_a, ssem_b]

    def load(ci):
        b = ci % 2
        pt0 = wid * (SD_NCHUNK * SD_PTS) + ci * SD_PTS
        cps = [pltpu.async_copy(pv_ref.at[pl.ds(pt0, SD_PTS)], pvb[b], lsem[b]),
               pltpu.async_copy(w_ref.at[pl.ds(pt0 * K, SD_PAIRS)], wb[b],
                                lsem[b])]
        cps += [
            pltpu.async_copy(idx_ref.at[pl.ds(pt0 * K + si * 128, 128)],
                             idxb[ci % 3].at[si], lsem[b])
            for si in range(K)
        ]
        return cps

    loads = [None, None]
    scats = [None, None]
    loads[0] = load(0)
    for ci in range(SD_NCHUNK):
        b = ci % 2
        if ci + 1 < SD_NCHUNK:
            loads[1 - b] = load(ci + 1)
        for cp in loads[b]:
            cp.wait()
        if scats[b] is not None:
            for cp in scats[b]:
                cp.wait()

        def point(j, carry2, _pvb=pvb[b], _wb=wb[b], _payl=payl[b]):
            pv0 = _pvb[j, pl.ds(0, 16)]
            pv1 = _pvb[j, pl.ds(16, 16)]
            pv2 = _pvb[j, pl.ds(32, 16)]
            for k in range(K):
                wv = plsc.load_gather(
                    _wb, [jnp.full((16,), j * K + k, jnp.int32)])
                _payl[j * K + k, pl.ds(0, 16)] = wv * pv0
                _payl[j * K + k, pl.ds(16, 16)] = wv * pv1
                _payl[j * K + k, pl.ds(32, 16)] = wv * pv2
            return carry2

        lax.fori_loop(0, SD_PTS, point, 0)
        scats[b] = [
            pltpu.async_copy(payl[b].at[pl.ds(si * 128, 128)],
                             acc.at[idxb[ci % 3].at[si]], ssem[b], add=True)
            for si in range(K)
        ]
    for pend in scats:
        if pend is not None:
            for cp in pend:
                cp.wait()
    plsc.subcore_barrier()

    @pl.when(s == 0)
    def _():
        pltpu.sync_copy(acc, out_ref.at[c])


def _sc_scatter(w_flat, idx2d, pv):
    mesh = plsc.VectorSubcoreMesh(core_axis_name="c", subcore_axis_name="s")
    f = functools.partial(
        pl.kernel,
        mesh=mesh,
        out_type=jax.ShapeDtypeStruct((2, MA, 48), jnp.float32),
        scratch_types=[
            pltpu.VMEM((SD_PTS, 48), jnp.float32),
            pltpu.VMEM((SD_PTS, 48), jnp.float32),
            pltpu.VMEM((SD_PAIRS,), jnp.float32),
            pltpu.VMEM((SD_PAIRS,), jnp.float32),
            pltpu.VMEM((K, 128), jnp.int32),
            pltpu.VMEM((K, 128), jnp.int32),
            pltpu.VMEM((K, 128), jnp.int32),
            pltpu.VMEM((SD_PAIRS, 48), jnp.float32),
            pltpu.VMEM((SD_PAIRS, 48), jnp.float32),
            pltpu.VMEM((MA // 16, 48), jnp.float32),
            pltpu.VMEM_SHARED((MA, 48), jnp.float32),
            pltpu.SemaphoreType.DMA,
            pltpu.SemaphoreType.DMA,
            pltpu.SemaphoreType.DMA,
            pltpu.SemaphoreType.DMA,
        ],
        compiler_params=pltpu.CompilerParams(use_tc_tiling_on_sc=False,
                                             needs_layout_passes=False),
    )(_scatter_body)
    return f(w_flat, idx2d, pv)


def kernel(sp_fea, sp_xyz, o_p_fea, p_xyz, c2p_idx_abs, c2p_idx, cluster_idx,
           mlp_W1, mlp_g1, mlp_b1, mlp_W2, mlp_g2, mlp_b2,
           wfea_W1, wfea_g1, wfea_b1, wfea_W2, wfea_g2, wfea_b2,
           wxyz_W1, wxyz_g1, wxyz_b1, wxyz_W2, wxyz_g2, wxyz_b2):
    f32 = jnp.float32
    spf, spx = sp_fea[0], sp_xyz[0]
    opf, pxy = o_p_fea[0], p_xyz[0]

    pad_pairs = NPK - N * K
    idxg = jnp.concatenate(
        [c2p_idx_abs[0].reshape(-1).astype(jnp.int32),
         jnp.zeros((pad_pairs,), jnp.int32)])
    idxs = jnp.concatenate(
        [c2p_idx[0].reshape(-1).astype(jnp.int32),
         jnp.zeros((pad_pairs,), jnp.int32)])

    pv = jnp.concatenate(
        [opf, pxy, jnp.ones((N, 1), f32), jnp.zeros((N, 12), f32)], axis=1)
    pv = jnp.concatenate([pv, jnp.zeros((NP - N, 48), f32)], axis=0)

    wm = jnp.zeros((48, H), f32).at[0:C].set(mlp_W1.T)
    wfp = jnp.zeros((48, H), f32).at[0:C].set(wfea_W1.T)
    wxp = jnp.zeros((48, H), f32).at[32:35].set(wxyz_W1.T)
    r = lambda v: v.reshape(1, H)

    KC = K * C
    w1big = jnp.zeros((48, KC), f32)
    w2big = jnp.zeros((KC, KC), f32)
    sel = jnp.zeros((KC, 2 * K), f32)
    for k in range(K):
        w1big = w1big.at[:, k * 32:k * 32 + 16].set(wfp)
        w1big = w1big.at[:, k * 32 + 16:k * 32 + 32].set(wxp)
        w2big = w2big.at[k * 32:k * 32 + 16, k * 32:k * 32 + 16].set(wfea_W2.T)
        w2big = w2big.at[k * 32 + 16:k * 32 + 32,
                         k * 32 + 16:k * 32 + 32].set(wxyz_W2.T)
        sel = sel.at[k * 32:k * 32 + 16, k].set(1.0)
        sel = sel.at[k * 32 + 16:k * 32 + 32, K + k].set(1.0)
    gx1 = jnp.concatenate([wfea_g1, wxyz_g1])
    bx1 = jnp.concatenate([wfea_b1, wxyz_b1])
    gx2 = jnp.concatenate([wfea_g2, wxyz_g2])
    bx2 = jnp.concatenate([wfea_b2, wxyz_b2])
    tile6 = lambda v: jnp.tile(v, (K,)).reshape(1, KC)

    tables = _project_tables(spf, spx, wfea_W1.T, wxyz_W1.T)
    g = _sc_gather(tables.reshape(-1), idxg)
    w = _dense_stage(
        g.reshape(NP, K * C), pv,
        wm, r(mlp_g1), r(mlp_b1), mlp_W2.T, r(mlp_g2), r(mlp_b2),
        w1big, tile6(gx1), tile6(bx1), w2big, tile6(gx2), tile6(bx2), sel)
    part = _sc_scatter(w.reshape(-1), idxs, pv)

    acc = part[0, :M] + part[1, :M]
    denom = acc[:, 35:36] + 1e-8
    return (acc[:, 0:C] / denom)[None], (acc[:, 32:35] / denom)[None]
